# baseline (speedup 1.0000x reference)
import jax
import jax.numpy as jnp
from jax.experimental import pallas as pl

_N = 50000
_Y = 64
_EMB = 64
_REG = 32
_AUX = 4
_NPY = _N + _Y
_NB = 8
_BLK = 6272
_NPAD = _NB * _BLK

_HI = jax.lax.Precision.HIGHEST
_INTERPRET = False


def _const_vecs(w_ref, p_ref):
    w = w_ref[...]
    c = jnp.maximum(w[0:1, :] + w[1:2, :], 0.0)
    c = c / jnp.maximum(jnp.sqrt(jnp.sum(c * c)), 1e-12)
    p = p_ref[...]
    pb = p.astype(jnp.bfloat16).astype(jnp.float32)
    u = jnp.dot(c, p, precision=_HI, preferred_element_type=jnp.float32)
    v = jnp.dot(c, pb, precision=_HI, preferred_element_type=jnp.float32)
    return c, u, v


def _node_rows_bf16(s, t, c, v, p_ref, ws_ref):
    x = (s * c).astype(jnp.bfloat16)
    xl = jnp.dot(x, p_ref[...].astype(jnp.bfloat16),
                 preferred_element_type=jnp.float32)
    neigh = t * v
    cc = jnp.concatenate([xl.astype(jnp.bfloat16),
                          neigh.astype(jnp.bfloat16)], axis=1)
    msg = jnp.maximum(
        jnp.dot(cc, ws_ref[...].astype(jnp.bfloat16),
                preferred_element_type=jnp.float32), 0.0)
    return msg


def _rownorm(msg):
    nrm = jnp.sqrt(jnp.sum(msg * msg, axis=1, keepdims=True))
    return msg / jnp.maximum(nrm, 1e-12)


def _dense_body(s_ref, ts_ref, c_ref, w_ref, p_ref, ws_ref, out_ref):
    c, _, v = _const_vecs(w_ref, p_ref)
    s = s_ref[0]
    t = ts_ref[0] / jnp.maximum(c_ref[0], 1.0)
    msg = _node_rows_bf16(s, t, c, v, p_ref, ws_ref)
    out_ref[0] = _rownorm(msg)


def _head_body(sy_ref, tsy_ref, cy_ref, sa_ref, tsa_ref, ca_ref, av_ref,
               aux_ref, w_ref, p_ref, ws_ref, h1_ref, h2_ref, cp_ref, q_ref):
    c, u, v = _const_vecs(w_ref, p_ref)
    wsb = ws_ref[...].astype(jnp.bfloat16)
    q = jnp.zeros((_Y, 1), jnp.float32)
    for l in range(2):
        sy = sy_ref[l]
        ty = tsy_ref[l] / jnp.maximum(cy_ref[l], 1.0)
        xl_y = sy * u
        cc_y = jnp.concatenate([xl_y.astype(jnp.bfloat16),
                                (ty * v).astype(jnp.bfloat16)], axis=1)
        ymsg = jnp.maximum(
            jnp.dot(cc_y, wsb, preferred_element_type=jnp.float32), 0.0)
        ypot = _rownorm(ymsg)
        sa = sa_ref[l]
        ta = tsa_ref[l] / jnp.maximum(ca_ref[l], 1.0)
        amsg = _node_rows_bf16(sa, ta, c, v, p_ref, ws_ref)
        ae = _rownorm(amsg) * av_ref[l]
        cj = jnp.dot(ypot, cp_ref[...], precision=_HI,
                     preferred_element_type=jnp.float32)
        embed = ae * cj
        hidden = jnp.maximum(
            jnp.dot(embed, h1_ref[...], precision=_HI,
                    preferred_element_type=jnp.float32), 0.0)
        q = q + jnp.dot(hidden, h2_ref[0:_REG, :], precision=_HI,
                        preferred_element_type=jnp.float32)
        q = q + jnp.dot(aux_ref[l], h2_ref[_REG:, :], precision=_HI,
                        preferred_element_type=jnp.float32)
    q_ref[...] = q


def kernel(node_input, n2n_index0, n2n_value0, n2n_index1, n2n_value1,
           subg_index0, subg_value0, subg_index1, subg_value1,
           action_index0, action_value0, action_index1, action_value1,
           aux_input, sage_edge0, sage_edge1, w_n2l, p_node_conv, W_sage,
           h1_weight, h2_weight, cross_product):
    f32 = jnp.float32
    seg = jax.ops.segment_sum
    s0 = seg(n2n_value0, n2n_index0[0], num_segments=_N)
    s1 = seg(n2n_value1, n2n_index1[0], num_segments=_N)
    sy0 = seg(subg_value0, subg_index0[0], num_segments=_Y)
    sy1 = seg(subg_value1, subg_index1[0], num_segments=_Y)
    S = jnp.stack([jnp.concatenate([s0, sy0]), jnp.concatenate([s1, sy1])])

    def sage_sums(S_l, edge):
        src, dst = edge[0], edge[1]
        ts = seg(S_l[src], dst, num_segments=_NPY)
        cn = seg(jnp.ones((edge.shape[1],), f32), dst, num_segments=_NPY)
        return ts, cn

    ts0, cn0 = sage_sums(S[0], sage_edge0)
    ts1, cn1 = sage_sums(S[1], sage_edge1)
    TS = jnp.stack([ts0, ts1])
    CN = jnp.stack([cn0, cn1])

    pad = ((0, 0), (0, _NPAD - _NPY))
    Sp = jnp.pad(S, pad)[..., None]
    TSp = jnp.pad(TS, pad)[..., None]
    CNp = jnp.pad(CN, pad)[..., None]

    bspec_s = pl.BlockSpec((1, _BLK, 1), lambda l, j: (l, j, 0))

    def bspec_w(shape):
        return pl.BlockSpec(shape, lambda l, j: tuple(0 for _ in shape))

    cur_msg = pl.pallas_call(
        _dense_body,
        grid=(2, _NB),
        in_specs=[bspec_s, bspec_s, bspec_s,
                  bspec_w((2, _EMB)), bspec_w((_EMB, _EMB)),
                  bspec_w((2 * _EMB, _EMB))],
        out_specs=pl.BlockSpec((1, _BLK, _EMB), lambda l, j: (l, j, 0)),
        out_shape=jax.ShapeDtypeStruct((2, _N, _EMB), f32),
        interpret=_INTERPRET,
    )(Sp, TSp, CNp, w_n2l, p_node_conv, W_sage)

    Sy = S[:, _N:][..., None]
    TSy = TS[:, _N:][..., None]
    CNy = CN[:, _N:][..., None]
    cols = jnp.stack([action_index0[1], action_index1[1]])
    Sa = jnp.take_along_axis(S, cols, axis=1)[..., None]
    TSa = jnp.take_along_axis(TS, cols, axis=1)[..., None]
    CNa = jnp.take_along_axis(CN, cols, axis=1)[..., None]
    av = jnp.stack([action_value0, action_value1])[..., None]
    aux_t = jnp.transpose(aux_input, (1, 0, 2))

    q = pl.pallas_call(
        _head_body,
        out_shape=jax.ShapeDtypeStruct((_Y, 1), f32),
        interpret=_INTERPRET,
    )(Sy, TSy, CNy, Sa, TSa, CNa, av, aux_t, w_n2l, p_node_conv, W_sage,
      h1_weight, h2_weight, cross_product)
    return (q, cur_msg)


# speedup vs baseline: 24.5055x; 24.5055x over previous
import jax
import jax.numpy as jnp
from jax import lax
from jax.experimental import pallas as pl
from jax.experimental.pallas import tpu as pltpu
from jax.experimental.pallas import tpu_sc as plsc

_N = 50000
_Y = 64
_EMB = 64
_REG = 32
_AUX = 4
_NPY = _N + _Y
_NB = 8
_BLK = 6400
_NPAD = _NB * _BLK

_HI = jax.lax.Precision.HIGHEST
_INTERPRET = False

_LANES = 128
_CHW = 8
_EPR = 6272
_SGR = 512
_PADBIN = _NPY
_TPW = _NPAD // 16


def _sc_body(n2n_row, n2n_val, subg_row, subg_val, sage_src, sage_dst,
             zeros_hbm, s0_out, s1_out, ts0_out, ts1_out, cn0_out, cn1_out,
             idx_v, val_v, g_v, ones_v, acc_s, acc_ts, acc_cn):
    ci = lax.axis_index("c")
    sid = lax.axis_index("s")
    slc = pl.ds(sid * _TPW, _TPW)
    pltpu.sync_copy(zeros_hbm.at[slc], acc_s.at[slc])
    pltpu.sync_copy(zeros_hbm.at[slc], acc_ts.at[slc])
    pltpu.sync_copy(zeros_hbm.at[slc], acc_cn.at[slc])
    for k in range(_LANES // 16):
        ones_v[pl.ds(k * 16, 16)] = jnp.full((16,), 1.0, jnp.float32)
    plsc.subcore_barrier()

    ebase = sid * (_EPR // 16)
    sbase = sid * (_SGR // 16)

    def p1(j, carry):
        off = pl.multiple_of(ebase + j * _CHW, _CHW)
        pltpu.sync_copy(n2n_row.at[ci, pl.ds(off, _CHW)], idx_v)
        pltpu.sync_copy(n2n_val.at[ci, pl.ds(off, _CHW)], val_v)
        for k in range(_CHW):
            pltpu.sync_copy(val_v.at[k], acc_s.at[idx_v.at[k]], add=True)
        return carry

    lax.fori_loop(0, _EPR // (16 * _CHW), p1, 0)

    def p1b(j, carry):
        off = pl.multiple_of(sbase + j * _CHW, _CHW)
        pltpu.sync_copy(subg_row.at[ci, pl.ds(off, _CHW)], idx_v)
        pltpu.sync_copy(subg_val.at[ci, pl.ds(off, _CHW)], val_v)
        for k in range(_CHW):
            pltpu.sync_copy(val_v.at[k], acc_s.at[idx_v.at[k]], add=True)
        return carry

    lax.fori_loop(0, _SGR // (16 * _CHW), p1b, 0)
    plsc.subcore_barrier()

    def p2(j, carry):
        off = pl.multiple_of(ebase + j * _CHW, _CHW)
        pltpu.sync_copy(sage_src.at[ci, pl.ds(off, _CHW)], idx_v)
        for k in range(_CHW):
            pltpu.sync_copy(acc_s.at[idx_v.at[k]], g_v.at[k])
        pltpu.sync_copy(sage_dst.at[ci, pl.ds(off, _CHW)], idx_v)
        for k in range(_CHW):
            pltpu.sync_copy(g_v.at[k], acc_ts.at[idx_v.at[k]], add=True)
            pltpu.sync_copy(ones_v, acc_cn.at[idx_v.at[k]], add=True)
        return carry

    lax.fori_loop(0, _EPR // (16 * _CHW), p2, 0)
    plsc.subcore_barrier()

    @pl.when(ci == 0)
    def _():
        pltpu.sync_copy(acc_s.at[slc], s0_out.at[slc])
        pltpu.sync_copy(acc_ts.at[slc], ts0_out.at[slc])
        pltpu.sync_copy(acc_cn.at[slc], cn0_out.at[slc])

    @pl.when(ci == 1)
    def _():
        pltpu.sync_copy(acc_s.at[slc], s1_out.at[slc])
        pltpu.sync_copy(acc_ts.at[slc], ts1_out.at[slc])
        pltpu.sync_copy(acc_cn.at[slc], cn1_out.at[slc])


def _sc_segment_sums(n2n_row, n2n_val, subg_row, subg_val, sage_src, sage_dst,
                     zeros):
    f32 = jnp.float32
    out6 = [jax.ShapeDtypeStruct((_NPAD,), f32)] * 6
    scr = [
        pltpu.VMEM((_CHW, _LANES), jnp.int32),
        pltpu.VMEM((_CHW, _LANES), f32),
        pltpu.VMEM((_CHW, _LANES), f32),
        pltpu.VMEM((_LANES,), f32),
        pltpu.VMEM_SHARED((_NPAD,), f32),
        pltpu.VMEM_SHARED((_NPAD,), f32),
        pltpu.VMEM_SHARED((_NPAD,), f32),
    ]
    mesh = plsc.VectorSubcoreMesh(core_axis_name="c", subcore_axis_name="s")
    s0, s1, ts0, ts1, cn0, cn1 = pl.kernel(
        _sc_body, mesh=mesh, out_type=out6, scratch_types=scr)(
        n2n_row, n2n_val, subg_row, subg_val, sage_src, sage_dst, zeros)
    return (jnp.stack([s0, s1]), jnp.stack([ts0, ts1]),
            jnp.stack([cn0, cn1]))


def _const_vecs(w_ref, p_ref):
    w = w_ref[...]
    c = jnp.maximum(w[0:1, :] + w[1:2, :], 0.0)
    c = c / jnp.maximum(jnp.sqrt(jnp.sum(c * c)), 1e-12)
    p = p_ref[...]
    pb = p.astype(jnp.bfloat16).astype(jnp.float32)
    u = jnp.dot(c, p, precision=_HI, preferred_element_type=jnp.float32)
    v = jnp.dot(c, pb, precision=_HI, preferred_element_type=jnp.float32)
    return c, u, v


def _node_rows_bf16(s, t, c, v, p_ref, ws_ref):
    x = (s * c).astype(jnp.bfloat16)
    xl = jnp.dot(x, p_ref[...].astype(jnp.bfloat16),
                 preferred_element_type=jnp.float32)
    neigh = t * v
    cc = jnp.concatenate([xl.astype(jnp.bfloat16),
                          neigh.astype(jnp.bfloat16)], axis=1)
    msg = jnp.maximum(
        jnp.dot(cc, ws_ref[...].astype(jnp.bfloat16),
                preferred_element_type=jnp.float32), 0.0)
    return msg


def _rownorm(msg):
    nrm = jnp.sqrt(jnp.sum(msg * msg, axis=1, keepdims=True))
    return msg / jnp.maximum(nrm, 1e-12)


def _dense_body(s_ref, ts_ref, c_ref, w_ref, p_ref, ws_ref, out_ref):
    c, _, v = _const_vecs(w_ref, p_ref)
    s = s_ref[0]
    t = ts_ref[0] / jnp.maximum(c_ref[0], 1.0)
    msg = _node_rows_bf16(s, t, c, v, p_ref, ws_ref)
    out_ref[0] = _rownorm(msg)


def _head_body(sy_ref, tsy_ref, cy_ref, sa_ref, tsa_ref, ca_ref, av_ref,
               aux_ref, w_ref, p_ref, ws_ref, h1_ref, h2_ref, cp_ref, q_ref):
    c, u, v = _const_vecs(w_ref, p_ref)
    wsb = ws_ref[...].astype(jnp.bfloat16)
    q = jnp.zeros((_Y, 1), jnp.float32)
    for l in range(2):
        sy = sy_ref[l]
        ty = tsy_ref[l] / jnp.maximum(cy_ref[l], 1.0)
        xl_y = sy * u
        cc_y = jnp.concatenate([xl_y.astype(jnp.bfloat16),
                                (ty * v).astype(jnp.bfloat16)], axis=1)
        ymsg = jnp.maximum(
            jnp.dot(cc_y, wsb, preferred_element_type=jnp.float32), 0.0)
        ypot = _rownorm(ymsg)
        sa = sa_ref[l]
        ta = tsa_ref[l] / jnp.maximum(ca_ref[l], 1.0)
        amsg = _node_rows_bf16(sa, ta, c, v, p_ref, ws_ref)
        ae = _rownorm(amsg) * av_ref[l]
        cj = jnp.dot(ypot, cp_ref[...], precision=_HI,
                     preferred_element_type=jnp.float32)
        embed = ae * cj
        hidden = jnp.maximum(
            jnp.dot(embed, h1_ref[...], precision=_HI,
                    preferred_element_type=jnp.float32), 0.0)
        q = q + jnp.dot(hidden, h2_ref[0:_REG, :], precision=_HI,
                        preferred_element_type=jnp.float32)
        q = q + jnp.dot(aux_ref[l], h2_ref[_REG:, :], precision=_HI,
                        preferred_element_type=jnp.float32)
    q_ref[...] = q


def kernel(node_input, n2n_index0, n2n_value0, n2n_index1, n2n_value1,
           subg_index0, subg_value0, subg_index1, subg_value1,
           action_index0, action_value0, action_index1, action_value1,
           aux_input, sage_edge0, sage_edge1, w_n2l, p_node_conv, W_sage,
           h1_weight, h2_weight, cross_product):
    f32 = jnp.float32

    def prep(x, rows, padval):
        n = rows * _LANES - x.shape[0]
        return jnp.reshape(jnp.pad(x, (0, n), constant_values=padval),
                           (rows, _LANES))

    n2n_row = jnp.stack([prep(n2n_index0[0], _EPR, _PADBIN),
                         prep(n2n_index1[0], _EPR, _PADBIN)])
    n2n_val = jnp.stack([prep(n2n_value0, _EPR, 0),
                         prep(n2n_value1, _EPR, 0)])
    subg_row = jnp.stack([prep(subg_index0[0] + _N, _SGR, _PADBIN),
                          prep(subg_index1[0] + _N, _SGR, _PADBIN)])
    subg_val = jnp.stack([prep(subg_value0, _SGR, 0),
                          prep(subg_value1, _SGR, 0)])
    sage_src_s = jnp.stack([prep(sage_edge0[0], _EPR, 0),
                            prep(sage_edge1[0], _EPR, 0)])
    sage_dst_s = jnp.stack([prep(sage_edge0[1], _EPR, _PADBIN),
                            prep(sage_edge1[1], _EPR, _PADBIN)])
    zeros = jnp.zeros((_NPAD,), f32)

    S, TS, CN = _sc_segment_sums(n2n_row, n2n_val, subg_row, subg_val,
                                 sage_src_s, sage_dst_s, zeros)

    Sp = S[..., None]
    TSp = TS[..., None]
    CNp = CN[..., None]

    bspec_s = pl.BlockSpec((1, _BLK, 1), lambda l, j: (l, j, 0))

    def bspec_w(shape):
        return pl.BlockSpec(shape, lambda l, j: tuple(0 for _ in shape))

    cur_msg = pl.pallas_call(
        _dense_body,
        grid=(2, _NB),
        in_specs=[bspec_s, bspec_s, bspec_s,
                  bspec_w((2, _EMB)), bspec_w((_EMB, _EMB)),
                  bspec_w((2 * _EMB, _EMB))],
        out_specs=pl.BlockSpec((1, _BLK, _EMB), lambda l, j: (l, j, 0)),
        out_shape=jax.ShapeDtypeStruct((2, _N, _EMB), f32),
        interpret=_INTERPRET,
    )(Sp, TSp, CNp, w_n2l, p_node_conv, W_sage)

    Sy = S[:, _N:_NPY][..., None]
    TSy = TS[:, _N:_NPY][..., None]
    CNy = CN[:, _N:_NPY][..., None]
    cols = jnp.stack([action_index0[1], action_index1[1]])
    Sa = jnp.take_along_axis(S, cols, axis=1)[..., None]
    TSa = jnp.take_along_axis(TS, cols, axis=1)[..., None]
    CNa = jnp.take_along_axis(CN, cols, axis=1)[..., None]
    av = jnp.stack([action_value0, action_value1])[..., None]
    aux_t = jnp.transpose(aux_input, (1, 0, 2))

    q = pl.pallas_call(
        _head_body,
        out_shape=jax.ShapeDtypeStruct((_Y, 1), f32),
        interpret=_INTERPRET,
    )(Sy, TSy, CNy, Sa, TSa, CNa, av, aux_t, w_n2l, p_node_conv, W_sage,
      h1_weight, h2_weight, cross_product)
    return (q, cur_msg)


# speedup vs baseline: 29.8610x; 1.2185x over previous
import jax
import jax.numpy as jnp
from jax import lax
from jax.experimental import pallas as pl
from jax.experimental.pallas import tpu as pltpu
from jax.experimental.pallas import tpu_sc as plsc

_N = 50000
_Y = 64
_EMB = 64
_REG = 32
_AUX = 4
_NPY = _N + _Y
_NB = 8
_BLK = 6400
_NPAD = _NB * _BLK

_HI = jax.lax.Precision.HIGHEST
_INTERPRET = False

_LANES = 128
_CHW = 8
_EPT = 400
_EPR = 16 * _EPT
_EPR_A = _EPR + _CHW
_SGR = 512
_PADBIN = _NPY
_TPW = _NPAD // 16
_NCH = _EPT // _CHW


def _sc_body(n2n_row, n2n_val, subg_row, subg_val, sage_src, sage_dst,
             zeros_hbm, s0_out, s1_out, ts0_out, ts1_out, cn0_out, cn1_out,
             si_v, di_v, val_v, g_v, ones_v, acc_s, acc_ts, acc_cn,
             sem_ld, sem_st, sem_g):
    ci = lax.axis_index("c")
    sid = lax.axis_index("s")
    slc = pl.ds(sid * _TPW, _TPW)
    pltpu.sync_copy(zeros_hbm.at[slc], acc_s.at[slc])
    pltpu.sync_copy(zeros_hbm.at[slc], acc_ts.at[slc])
    pltpu.sync_copy(zeros_hbm.at[slc], acc_cn.at[slc])
    for k in range(_LANES // 16):
        ones_v[pl.ds(k * 16, 16)] = jnp.full((16,), 1.0, jnp.float32)
    plsc.subcore_barrier()

    ebase = sid * _EPT
    sbase = sid * (_SGR // 16)

    def echunk(hbm, ch, buf):
        off = pl.multiple_of(ebase + ch * _CHW, _CHW)
        return hbm.at[ci, pl.ds(off, _CHW)], buf

    pltpu.async_copy(*echunk(n2n_row, 0, si_v.at[0]), sem_ld)
    pltpu.async_copy(*echunk(n2n_val, 0, val_v.at[0]), sem_ld)

    def p1(j, carry):
        for b in range(2):
            ch = 2 * j + b
            pltpu.make_async_copy(*echunk(n2n_row, ch, si_v.at[b]), sem_ld).wait()
            pltpu.make_async_copy(*echunk(n2n_val, ch, val_v.at[b]), sem_ld).wait()
            for k in range(_CHW):
                pltpu.async_copy(val_v.at[b, k], acc_s.at[si_v.at[b, k]],
                                 sem_st, add=True)
            pltpu.async_copy(*echunk(n2n_row, ch + 1, si_v.at[1 - b]), sem_ld)
            pltpu.async_copy(*echunk(n2n_val, ch + 1, val_v.at[1 - b]), sem_ld)
            for k in range(_CHW):
                pltpu.make_async_copy(val_v.at[b, k], acc_s.at[si_v.at[b, k]],
                                      sem_st).wait()
        return carry

    lax.fori_loop(0, _NCH // 2, p1, 0)
    pltpu.make_async_copy(*echunk(n2n_row, _NCH, si_v.at[0]), sem_ld).wait()
    pltpu.make_async_copy(*echunk(n2n_val, _NCH, val_v.at[0]), sem_ld).wait()

    for j in range(_SGR // (16 * _CHW)):
        off = pl.multiple_of(sbase + j * _CHW, _CHW)
        pltpu.sync_copy(subg_row.at[ci, pl.ds(off, _CHW)], si_v.at[0])
        pltpu.sync_copy(subg_val.at[ci, pl.ds(off, _CHW)], val_v.at[0])
        for k in range(_CHW):
            pltpu.sync_copy(val_v.at[0, k], acc_s.at[si_v.at[0, k]], add=True)
    plsc.subcore_barrier()

    pltpu.async_copy(*echunk(sage_src, 0, si_v.at[0]), sem_ld)
    pltpu.async_copy(*echunk(sage_dst, 0, di_v.at[0]), sem_ld)

    def p2(j, carry):
        for b in range(2):
            ch = 2 * j + b
            pltpu.make_async_copy(*echunk(sage_src, ch, si_v.at[b]), sem_ld).wait()
            pltpu.make_async_copy(*echunk(sage_dst, ch, di_v.at[b]), sem_ld).wait()
            for k in range(_CHW):
                pltpu.async_copy(acc_s.at[si_v.at[b, k]], g_v.at[k], sem_g)
            for k in range(_CHW):
                pltpu.make_async_copy(acc_s.at[si_v.at[b, k]], g_v.at[k],
                                      sem_g).wait()
            for k in range(_CHW):
                pltpu.async_copy(g_v.at[k], acc_ts.at[di_v.at[b, k]],
                                 sem_st, add=True)
                pltpu.async_copy(ones_v, acc_cn.at[di_v.at[b, k]],
                                 sem_st, add=True)
            pltpu.async_copy(*echunk(sage_src, ch + 1, si_v.at[1 - b]), sem_ld)
            pltpu.async_copy(*echunk(sage_dst, ch + 1, di_v.at[1 - b]), sem_ld)
            for k in range(_CHW):
                pltpu.make_async_copy(g_v.at[k], acc_ts.at[di_v.at[b, k]],
                                      sem_st).wait()
                pltpu.make_async_copy(ones_v, acc_cn.at[di_v.at[b, k]],
                                      sem_st).wait()
        return carry

    lax.fori_loop(0, _NCH // 2, p2, 0)
    pltpu.make_async_copy(*echunk(sage_src, _NCH, si_v.at[0]), sem_ld).wait()
    pltpu.make_async_copy(*echunk(sage_dst, _NCH, di_v.at[0]), sem_ld).wait()
    plsc.subcore_barrier()

    @pl.when(ci == 0)
    def _():
        pltpu.sync_copy(acc_s.at[slc], s0_out.at[slc])
        pltpu.sync_copy(acc_ts.at[slc], ts0_out.at[slc])
        pltpu.sync_copy(acc_cn.at[slc], cn0_out.at[slc])

    @pl.when(ci == 1)
    def _():
        pltpu.sync_copy(acc_s.at[slc], s1_out.at[slc])
        pltpu.sync_copy(acc_ts.at[slc], ts1_out.at[slc])
        pltpu.sync_copy(acc_cn.at[slc], cn1_out.at[slc])


def _sc_segment_sums(n2n_row, n2n_val, subg_row, subg_val, sage_src, sage_dst,
                     zeros):
    f32 = jnp.float32
    out6 = [jax.ShapeDtypeStruct((_NPAD,), f32)] * 6
    scr = [
        pltpu.VMEM((2, _CHW, _LANES), jnp.int32),
        pltpu.VMEM((2, _CHW, _LANES), jnp.int32),
        pltpu.VMEM((2, _CHW, _LANES), f32),
        pltpu.VMEM((_CHW, _LANES), f32),
        pltpu.VMEM((_LANES,), f32),
        pltpu.VMEM_SHARED((_NPAD,), f32),
        pltpu.VMEM_SHARED((_NPAD,), f32),
        pltpu.VMEM_SHARED((_NPAD,), f32),
        pltpu.SemaphoreType.DMA,
        pltpu.SemaphoreType.DMA,
        pltpu.SemaphoreType.DMA,
    ]
    mesh = plsc.VectorSubcoreMesh(core_axis_name="c", subcore_axis_name="s")
    s0, s1, ts0, ts1, cn0, cn1 = pl.kernel(
        _sc_body, mesh=mesh, out_type=out6, scratch_types=scr)(
        n2n_row, n2n_val, subg_row, subg_val, sage_src, sage_dst, zeros)
    return (jnp.stack([s0, s1]), jnp.stack([ts0, ts1]),
            jnp.stack([cn0, cn1]))


def _const_vecs(w_ref, p_ref):
    w = w_ref[...]
    c = jnp.maximum(w[0:1, :] + w[1:2, :], 0.0)
    c = c / jnp.maximum(jnp.sqrt(jnp.sum(c * c)), 1e-12)
    p = p_ref[...]
    pb = p.astype(jnp.bfloat16).astype(jnp.float32)
    u = jnp.dot(c, p, precision=_HI, preferred_element_type=jnp.float32)
    v = jnp.dot(c, pb, precision=_HI, preferred_element_type=jnp.float32)
    return c, u, v


def _node_rows_bf16(s, t, c, v, p_ref, ws_ref):
    x = (s * c).astype(jnp.bfloat16)
    xl = jnp.dot(x, p_ref[...].astype(jnp.bfloat16),
                 preferred_element_type=jnp.float32)
    neigh = t * v
    cc = jnp.concatenate([xl.astype(jnp.bfloat16),
                          neigh.astype(jnp.bfloat16)], axis=1)
    msg = jnp.maximum(
        jnp.dot(cc, ws_ref[...].astype(jnp.bfloat16),
                preferred_element_type=jnp.float32), 0.0)
    return msg


def _rownorm(msg):
    nrm = jnp.sqrt(jnp.sum(msg * msg, axis=1, keepdims=True))
    return msg / jnp.maximum(nrm, 1e-12)


def _dense_body(s_ref, ts_ref, c_ref, w_ref, p_ref, ws_ref, out_ref):
    c, _, v = _const_vecs(w_ref, p_ref)
    s = s_ref[0]
    t = ts_ref[0] / jnp.maximum(c_ref[0], 1.0)
    msg = _node_rows_bf16(s, t, c, v, p_ref, ws_ref)
    out_ref[0] = _rownorm(msg)


def _head_body(sy_ref, tsy_ref, cy_ref, sa_ref, tsa_ref, ca_ref, av_ref,
               aux_ref, w_ref, p_ref, ws_ref, h1_ref, h2_ref, cp_ref, q_ref):
    c, u, v = _const_vecs(w_ref, p_ref)
    wsb = ws_ref[...].astype(jnp.bfloat16)
    q = jnp.zeros((_Y, 1), jnp.float32)
    for l in range(2):
        sy = sy_ref[l]
        ty = tsy_ref[l] / jnp.maximum(cy_ref[l], 1.0)
        xl_y = sy * u
        cc_y = jnp.concatenate([xl_y.astype(jnp.bfloat16),
                                (ty * v).astype(jnp.bfloat16)], axis=1)
        ymsg = jnp.maximum(
            jnp.dot(cc_y, wsb, preferred_element_type=jnp.float32), 0.0)
        ypot = _rownorm(ymsg)
        sa = sa_ref[l]
        ta = tsa_ref[l] / jnp.maximum(ca_ref[l], 1.0)
        amsg = _node_rows_bf16(sa, ta, c, v, p_ref, ws_ref)
        ae = _rownorm(amsg) * av_ref[l]
        cj = jnp.dot(ypot, cp_ref[...], precision=_HI,
                     preferred_element_type=jnp.float32)
        embed = ae * cj
        hidden = jnp.maximum(
            jnp.dot(embed, h1_ref[...], precision=_HI,
                    preferred_element_type=jnp.float32), 0.0)
        q = q + jnp.dot(hidden, h2_ref[0:_REG, :], precision=_HI,
                        preferred_element_type=jnp.float32)
        q = q + jnp.dot(aux_ref[l], h2_ref[_REG:, :], precision=_HI,
                        preferred_element_type=jnp.float32)
    q_ref[...] = q


def kernel(node_input, n2n_index0, n2n_value0, n2n_index1, n2n_value1,
           subg_index0, subg_value0, subg_index1, subg_value1,
           action_index0, action_value0, action_index1, action_value1,
           aux_input, sage_edge0, sage_edge1, w_n2l, p_node_conv, W_sage,
           h1_weight, h2_weight, cross_product):
    f32 = jnp.float32

    def prep(x, rows, padval):
        n = rows * _LANES - x.shape[0]
        return jnp.reshape(jnp.pad(x, (0, n), constant_values=padval),
                           (rows, _LANES))

    n2n_row = jnp.stack([prep(n2n_index0[0], _EPR_A, _PADBIN),
                         prep(n2n_index1[0], _EPR_A, _PADBIN)])
    n2n_val = jnp.stack([prep(n2n_value0, _EPR_A, 0),
                         prep(n2n_value1, _EPR_A, 0)])
    subg_row = jnp.stack([prep(subg_index0[0] + _N, _SGR, _PADBIN),
                          prep(subg_index1[0] + _N, _SGR, _PADBIN)])
    subg_val = jnp.stack([prep(subg_value0, _SGR, 0),
                          prep(subg_value1, _SGR, 0)])
    sage_src_s = jnp.stack([prep(sage_edge0[0], _EPR_A, 0),
                            prep(sage_edge1[0], _EPR_A, 0)])
    sage_dst_s = jnp.stack([prep(sage_edge0[1], _EPR_A, _PADBIN),
                            prep(sage_edge1[1], _EPR_A, _PADBIN)])
    zeros = jnp.zeros((_NPAD,), f32)

    S, TS, CN = _sc_segment_sums(n2n_row, n2n_val, subg_row, subg_val,
                                 sage_src_s, sage_dst_s, zeros)

    Sp = S[..., None]
    TSp = TS[..., None]
    CNp = CN[..., None]

    bspec_s = pl.BlockSpec((1, _BLK, 1), lambda l, j: (l, j, 0))

    def bspec_w(shape):
        return pl.BlockSpec(shape, lambda l, j: tuple(0 for _ in shape))

    cur_msg = pl.pallas_call(
        _dense_body,
        grid=(2, _NB),
        in_specs=[bspec_s, bspec_s, bspec_s,
                  bspec_w((2, _EMB)), bspec_w((_EMB, _EMB)),
                  bspec_w((2 * _EMB, _EMB))],
        out_specs=pl.BlockSpec((1, _BLK, _EMB), lambda l, j: (l, j, 0)),
        out_shape=jax.ShapeDtypeStruct((2, _N, _EMB), f32),
        interpret=_INTERPRET,
    )(Sp, TSp, CNp, w_n2l, p_node_conv, W_sage)

    Sy = S[:, _N:_NPY][..., None]
    TSy = TS[:, _N:_NPY][..., None]
    CNy = CN[:, _N:_NPY][..., None]
    cols = jnp.stack([action_index0[1], action_index1[1]])
    Sa = jnp.take_along_axis(S, cols, axis=1)[..., None]
    TSa = jnp.take_along_axis(TS, cols, axis=1)[..., None]
    CNa = jnp.take_along_axis(CN, cols, axis=1)[..., None]
    av = jnp.stack([action_value0, action_value1])[..., None]
    aux_t = jnp.transpose(aux_input, (1, 0, 2))

    q = pl.pallas_call(
        _head_body,
        out_shape=jax.ShapeDtypeStruct((_Y, 1), f32),
        interpret=_INTERPRET,
    )(Sy, TSy, CNy, Sa, TSa, CNa, av, aux_t, w_n2l, p_node_conv, W_sage,
      h1_weight, h2_weight, cross_product)
    return (q, cur_msg)


# speedup vs baseline: 29.8805x; 1.0007x over previous
import jax
import jax.numpy as jnp
from jax import lax
from jax.experimental import pallas as pl
from jax.experimental.pallas import tpu as pltpu
from jax.experimental.pallas import tpu_sc as plsc

_N = 50000
_Y = 64
_EMB = 64
_REG = 32
_AUX = 4
_NPY = _N + _Y
_NB = 8
_BLK = 6400
_NPAD = _NB * _BLK

_HI = jax.lax.Precision.HIGHEST
_INTERPRET = False

_LANES = 128
_CHW = 8
_EPT = 400
_EPR = 16 * _EPT
_EPR_A = _EPR + _CHW
_SGR = 512
_PADBIN = _NPY
_TPW = _NPAD // 16
_NCH = _EPT // _CHW


def _sc_body(n2n_row, n2n_val, subg_row, subg_val, sage_src, sage_dst,
             zeros_hbm, acols, s0_out, s1_out, ts0_out, ts1_out, cn0_out,
             cn1_out, sa0_out, sa1_out, tsa0_out, tsa1_out, ca0_out, ca1_out,
             si_v, di_v, val_v, g_v, ones_v, ac_v, sa_v, tsa_v, ca_v,
             acc_s, acc_ts, acc_cn, sem_ld, sem_st, sem_g):
    ci = lax.axis_index("c")
    sid = lax.axis_index("s")
    slc = pl.ds(sid * _TPW, _TPW)
    pltpu.sync_copy(zeros_hbm.at[slc], acc_s.at[slc])
    pltpu.sync_copy(zeros_hbm.at[slc], acc_ts.at[slc])
    pltpu.sync_copy(zeros_hbm.at[slc], acc_cn.at[slc])
    for k in range(_LANES // 16):
        ones_v[pl.ds(k * 16, 16)] = jnp.full((16,), 1.0, jnp.float32)
    plsc.subcore_barrier()

    ebase = sid * _EPT
    sbase = sid * (_SGR // 16)

    def echunk(hbm, ch, buf):
        off = pl.multiple_of(ebase + ch * _CHW, _CHW)
        return hbm.at[ci, pl.ds(off, _CHW)], buf

    pltpu.async_copy(*echunk(n2n_row, 0, si_v.at[0]), sem_ld)
    pltpu.async_copy(*echunk(n2n_val, 0, val_v.at[0]), sem_ld)

    def p1(j, carry):
        for b in range(2):
            ch = 2 * j + b
            pltpu.make_async_copy(*echunk(n2n_row, ch, si_v.at[b]), sem_ld).wait()
            pltpu.make_async_copy(*echunk(n2n_val, ch, val_v.at[b]), sem_ld).wait()
            for k in range(_CHW):
                pltpu.async_copy(val_v.at[b, k], acc_s.at[si_v.at[b, k]],
                                 sem_st, add=True)
            pltpu.async_copy(*echunk(n2n_row, ch + 1, si_v.at[1 - b]), sem_ld)
            pltpu.async_copy(*echunk(n2n_val, ch + 1, val_v.at[1 - b]), sem_ld)
            for k in range(_CHW):
                pltpu.make_async_copy(val_v.at[b, k], acc_s.at[si_v.at[b, k]],
                                      sem_st).wait()
        return carry

    lax.fori_loop(0, _NCH // 2, p1, 0)
    pltpu.make_async_copy(*echunk(n2n_row, _NCH, si_v.at[0]), sem_ld).wait()
    pltpu.make_async_copy(*echunk(n2n_val, _NCH, val_v.at[0]), sem_ld).wait()

    for j in range(_SGR // (16 * _CHW)):
        off = pl.multiple_of(sbase + j * _CHW, _CHW)
        pltpu.sync_copy(subg_row.at[ci, pl.ds(off, _CHW)], si_v.at[0])
        pltpu.sync_copy(subg_val.at[ci, pl.ds(off, _CHW)], val_v.at[0])
        for k in range(_CHW):
            pltpu.sync_copy(val_v.at[0, k], acc_s.at[si_v.at[0, k]], add=True)
    plsc.subcore_barrier()

    pltpu.async_copy(*echunk(sage_src, 0, si_v.at[0]), sem_ld)
    pltpu.async_copy(*echunk(sage_dst, 0, di_v.at[0]), sem_ld)

    def p2(j, carry):
        for b in range(2):
            ch = 2 * j + b
            pltpu.make_async_copy(*echunk(sage_src, ch, si_v.at[b]), sem_ld).wait()
            pltpu.make_async_copy(*echunk(sage_dst, ch, di_v.at[b]), sem_ld).wait()
            for k in range(_CHW):
                pltpu.async_copy(acc_s.at[si_v.at[b, k]], g_v.at[k], sem_g)
            for k in range(_CHW):
                pltpu.make_async_copy(acc_s.at[si_v.at[b, k]], g_v.at[k],
                                      sem_g).wait()
            for k in range(_CHW):
                pltpu.async_copy(g_v.at[k], acc_ts.at[di_v.at[b, k]],
                                 sem_st, add=True)
                pltpu.async_copy(ones_v, acc_cn.at[di_v.at[b, k]],
                                 sem_st, add=True)
            pltpu.async_copy(*echunk(sage_src, ch + 1, si_v.at[1 - b]), sem_ld)
            pltpu.async_copy(*echunk(sage_dst, ch + 1, di_v.at[1 - b]), sem_ld)
            for k in range(_CHW):
                pltpu.make_async_copy(g_v.at[k], acc_ts.at[di_v.at[b, k]],
                                      sem_st).wait()
                pltpu.make_async_copy(ones_v, acc_cn.at[di_v.at[b, k]],
                                      sem_st).wait()
        return carry

    lax.fori_loop(0, _NCH // 2, p2, 0)
    pltpu.make_async_copy(*echunk(sage_src, _NCH, si_v.at[0]), sem_ld).wait()
    pltpu.make_async_copy(*echunk(sage_dst, _NCH, di_v.at[0]), sem_ld).wait()
    plsc.subcore_barrier()

    @pl.when(ci == 0)
    def _():
        pltpu.sync_copy(acc_s.at[slc], s0_out.at[slc])
        pltpu.sync_copy(acc_ts.at[slc], ts0_out.at[slc])
        pltpu.sync_copy(acc_cn.at[slc], cn0_out.at[slc])

    @pl.when(ci == 1)
    def _():
        pltpu.sync_copy(acc_s.at[slc], s1_out.at[slc])
        pltpu.sync_copy(acc_ts.at[slc], ts1_out.at[slc])
        pltpu.sync_copy(acc_cn.at[slc], cn1_out.at[slc])

    @pl.when(sid == 0)
    def _():
        pltpu.sync_copy(acols.at[pl.ds(ci * _LANES, _LANES)], ac_v)
        pltpu.sync_copy(acc_s.at[ac_v], sa_v)
        pltpu.sync_copy(acc_ts.at[ac_v], tsa_v)
        pltpu.sync_copy(acc_cn.at[ac_v], ca_v)

        @pl.when(ci == 0)
        def _():
            pltpu.sync_copy(sa_v, sa0_out)
            pltpu.sync_copy(tsa_v, tsa0_out)
            pltpu.sync_copy(ca_v, ca0_out)

        @pl.when(ci == 1)
        def _():
            pltpu.sync_copy(sa_v, sa1_out)
            pltpu.sync_copy(tsa_v, tsa1_out)
            pltpu.sync_copy(ca_v, ca1_out)


def _sc_segment_sums(n2n_row, n2n_val, subg_row, subg_val, sage_src, sage_dst,
                     zeros, acols):
    f32 = jnp.float32
    outs = ([jax.ShapeDtypeStruct((_NPAD,), f32)] * 6
            + [jax.ShapeDtypeStruct((_LANES,), f32)] * 6)
    scr = [
        pltpu.VMEM((2, _CHW, _LANES), jnp.int32),
        pltpu.VMEM((2, _CHW, _LANES), jnp.int32),
        pltpu.VMEM((2, _CHW, _LANES), f32),
        pltpu.VMEM((_CHW, _LANES), f32),
        pltpu.VMEM((_LANES,), f32),
        pltpu.VMEM((_LANES,), jnp.int32),
        pltpu.VMEM((_LANES,), f32),
        pltpu.VMEM((_LANES,), f32),
        pltpu.VMEM((_LANES,), f32),
        pltpu.VMEM_SHARED((_NPAD,), f32),
        pltpu.VMEM_SHARED((_NPAD,), f32),
        pltpu.VMEM_SHARED((_NPAD,), f32),
        pltpu.SemaphoreType.DMA,
        pltpu.SemaphoreType.DMA,
        pltpu.SemaphoreType.DMA,
    ]
    mesh = plsc.VectorSubcoreMesh(core_axis_name="c", subcore_axis_name="s")
    s0, s1, ts0, ts1, cn0, cn1, sa0, sa1, tsa0, tsa1, ca0, ca1 = pl.kernel(
        _sc_body, mesh=mesh, out_type=outs, scratch_types=scr)(
        n2n_row, n2n_val, subg_row, subg_val, sage_src, sage_dst, zeros, acols)
    return (jnp.stack([s0, s1]), jnp.stack([ts0, ts1]),
            jnp.stack([cn0, cn1]), jnp.stack([sa0, sa1]),
            jnp.stack([tsa0, tsa1]), jnp.stack([ca0, ca1]))


def _const_vecs(w_ref, p_ref):
    w = w_ref[...]
    c = jnp.maximum(w[0:1, :] + w[1:2, :], 0.0)
    c = c / jnp.maximum(jnp.sqrt(jnp.sum(c * c)), 1e-12)
    p = p_ref[...]
    pb = p.astype(jnp.bfloat16).astype(jnp.float32)
    u = jnp.dot(c, p, precision=_HI, preferred_element_type=jnp.float32)
    v = jnp.dot(c, pb, precision=_HI, preferred_element_type=jnp.float32)
    return c, u, v


def _node_rows_bf16(s, t, c, v, p_ref, ws_ref):
    x = (s * c).astype(jnp.bfloat16)
    xl = jnp.dot(x, p_ref[...].astype(jnp.bfloat16),
                 preferred_element_type=jnp.float32)
    neigh = t * v
    cc = jnp.concatenate([xl.astype(jnp.bfloat16),
                          neigh.astype(jnp.bfloat16)], axis=1)
    msg = jnp.maximum(
        jnp.dot(cc, ws_ref[...].astype(jnp.bfloat16),
                preferred_element_type=jnp.float32), 0.0)
    return msg


def _rownorm(msg):
    nrm = jnp.sqrt(jnp.sum(msg * msg, axis=1, keepdims=True))
    return msg / jnp.maximum(nrm, 1e-12)


def _dense_body(s_ref, ts_ref, c_ref, w_ref, p_ref, ws_ref, out_ref):
    c, _, v = _const_vecs(w_ref, p_ref)
    s = s_ref[0]
    t = ts_ref[0] / jnp.maximum(c_ref[0], 1.0)
    msg = _node_rows_bf16(s, t, c, v, p_ref, ws_ref)
    out_ref[0] = _rownorm(msg)


def _head_body(sy_ref, tsy_ref, cy_ref, sa_ref, tsa_ref, ca_ref, av_ref,
               aux_ref, w_ref, p_ref, ws_ref, h1_ref, h2_ref, cp_ref, q_ref):
    c, u, v = _const_vecs(w_ref, p_ref)
    wsb = ws_ref[...].astype(jnp.bfloat16)
    q = jnp.zeros((_Y, 1), jnp.float32)
    for l in range(2):
        sy = sy_ref[l]
        ty = tsy_ref[l] / jnp.maximum(cy_ref[l], 1.0)
        xl_y = sy * u
        cc_y = jnp.concatenate([xl_y.astype(jnp.bfloat16),
                                (ty * v).astype(jnp.bfloat16)], axis=1)
        ymsg = jnp.maximum(
            jnp.dot(cc_y, wsb, preferred_element_type=jnp.float32), 0.0)
        ypot = _rownorm(ymsg)
        sa = sa_ref[l]
        ta = tsa_ref[l] / jnp.maximum(ca_ref[l], 1.0)
        amsg = _node_rows_bf16(sa, ta, c, v, p_ref, ws_ref)
        ae = _rownorm(amsg) * av_ref[l]
        cj = jnp.dot(ypot, cp_ref[...], precision=_HI,
                     preferred_element_type=jnp.float32)
        embed = ae * cj
        hidden = jnp.maximum(
            jnp.dot(embed, h1_ref[...], precision=_HI,
                    preferred_element_type=jnp.float32), 0.0)
        q = q + jnp.dot(hidden, h2_ref[0:_REG, :], precision=_HI,
                        preferred_element_type=jnp.float32)
        q = q + jnp.dot(aux_ref[l], h2_ref[_REG:, :], precision=_HI,
                        preferred_element_type=jnp.float32)
    q_ref[...] = q


def kernel(node_input, n2n_index0, n2n_value0, n2n_index1, n2n_value1,
           subg_index0, subg_value0, subg_index1, subg_value1,
           action_index0, action_value0, action_index1, action_value1,
           aux_input, sage_edge0, sage_edge1, w_n2l, p_node_conv, W_sage,
           h1_weight, h2_weight, cross_product):
    f32 = jnp.float32

    def prep(x, rows, padval):
        n = rows * _LANES - x.shape[0]
        return jnp.reshape(jnp.pad(x, (0, n), constant_values=padval),
                           (rows, _LANES))

    n2n_row = jnp.stack([prep(n2n_index0[0], _EPR_A, _PADBIN),
                         prep(n2n_index1[0], _EPR_A, _PADBIN)])
    n2n_val = jnp.stack([prep(n2n_value0, _EPR_A, 0),
                         prep(n2n_value1, _EPR_A, 0)])
    subg_row = jnp.stack([prep(subg_index0[0] + _N, _SGR, _PADBIN),
                          prep(subg_index1[0] + _N, _SGR, _PADBIN)])
    subg_val = jnp.stack([prep(subg_value0, _SGR, 0),
                          prep(subg_value1, _SGR, 0)])
    sage_src_s = jnp.stack([prep(sage_edge0[0], _EPR_A, 0),
                            prep(sage_edge1[0], _EPR_A, 0)])
    sage_dst_s = jnp.stack([prep(sage_edge0[1], _EPR_A, _PADBIN),
                            prep(sage_edge1[1], _EPR_A, _PADBIN)])
    zeros = jnp.zeros((_NPAD,), f32)
    acols = jnp.concatenate([
        jnp.pad(action_index0[1], (0, _LANES - _Y)),
        jnp.pad(action_index1[1], (0, _LANES - _Y))])

    S, TS, CN, SaG, TSaG, CNaG = _sc_segment_sums(
        n2n_row, n2n_val, subg_row, subg_val, sage_src_s, sage_dst_s, zeros,
        acols)

    Sp = S[..., None]
    TSp = TS[..., None]
    CNp = CN[..., None]

    bspec_s = pl.BlockSpec((1, _BLK, 1), lambda l, j: (l, j, 0))

    def bspec_w(shape):
        return pl.BlockSpec(shape, lambda l, j: tuple(0 for _ in shape))

    cur_msg = pl.pallas_call(
        _dense_body,
        grid=(2, _NB),
        in_specs=[bspec_s, bspec_s, bspec_s,
                  bspec_w((2, _EMB)), bspec_w((_EMB, _EMB)),
                  bspec_w((2 * _EMB, _EMB))],
        out_specs=pl.BlockSpec((1, _BLK, _EMB), lambda l, j: (l, j, 0)),
        out_shape=jax.ShapeDtypeStruct((2, _N, _EMB), f32),
        interpret=_INTERPRET,
    )(Sp, TSp, CNp, w_n2l, p_node_conv, W_sage)

    Sy = S[:, _N:_NPY][..., None]
    TSy = TS[:, _N:_NPY][..., None]
    CNy = CN[:, _N:_NPY][..., None]
    Sa = SaG[:, :_Y][..., None]
    TSa = TSaG[:, :_Y][..., None]
    CNa = CNaG[:, :_Y][..., None]
    av = jnp.stack([action_value0, action_value1])[..., None]
    aux_t = jnp.transpose(aux_input, (1, 0, 2))

    q = pl.pallas_call(
        _head_body,
        out_shape=jax.ShapeDtypeStruct((_Y, 1), f32),
        interpret=_INTERPRET,
    )(Sy, TSy, CNy, Sa, TSa, CNa, av, aux_t, w_n2l, p_node_conv, W_sage,
      h1_weight, h2_weight, cross_product)
    return (q, cur_msg)


# speedup vs baseline: 29.8913x; 1.0004x over previous
import jax
import jax.numpy as jnp
from jax import lax
from jax.experimental import pallas as pl
from jax.experimental.pallas import tpu as pltpu
from jax.experimental.pallas import tpu_sc as plsc

_N = 50000
_Y = 64
_EMB = 64
_REG = 32
_AUX = 4
_NPY = _N + _Y
_NB = 8
_BLK = 6400
_NPAD = _NB * _BLK

_HI = jax.lax.Precision.HIGHEST

_LANES = 128
_CHW = 8
_EPT = 400
_EPR = 16 * _EPT
_EPR_A = _EPR + _CHW
_SGR = 512
_PADBIN = _NPY
_TPW = _NPAD // 16
_NCH = _EPT // _CHW


def _sc_body(n2n_row, n2n_val, subg_row, subg_val, sage_src, sage_dst,
             zeros_hbm, acols, s0_out, s1_out, ts0_out, ts1_out, cn0_out,
             cn1_out, sa0_out, sa1_out, tsa0_out, tsa1_out, ca0_out, ca1_out,
             si_v, di_v, val_v, g_v, ones_v, ac_v, sa_v, tsa_v, ca_v,
             acc_s, acc_ts, acc_cn, sem_ld, sem_st, sem_g):
    ci = lax.axis_index("c")
    sid = lax.axis_index("s")
    slc = pl.ds(sid * _TPW, _TPW)
    pltpu.sync_copy(zeros_hbm.at[slc], acc_s.at[slc])
    pltpu.sync_copy(zeros_hbm.at[slc], acc_ts.at[slc])
    pltpu.sync_copy(zeros_hbm.at[slc], acc_cn.at[slc])
    for k in range(_LANES // 16):
        ones_v[pl.ds(k * 16, 16)] = jnp.full((16,), 1.0, jnp.float32)
    plsc.subcore_barrier()

    ebase = sid * _EPT
    sbase = sid * (_SGR // 16)

    def echunk(hbm, ch, buf):
        off = pl.multiple_of(ebase + ch * _CHW, _CHW)
        return hbm.at[ci, pl.ds(off, _CHW)], buf

    pltpu.async_copy(*echunk(n2n_row, 0, si_v.at[0]), sem_ld)
    pltpu.async_copy(*echunk(n2n_val, 0, val_v.at[0]), sem_ld)

    def p1(j, carry):
        for b in range(2):
            ch = 2 * j + b
            pltpu.make_async_copy(*echunk(n2n_row, ch, si_v.at[b]), sem_ld).wait()
            pltpu.make_async_copy(*echunk(n2n_val, ch, val_v.at[b]), sem_ld).wait()
            for k in range(_CHW):
                pltpu.async_copy(val_v.at[b, k], acc_s.at[si_v.at[b, k]],
                                 sem_st, add=True)
            pltpu.async_copy(*echunk(n2n_row, ch + 1, si_v.at[1 - b]), sem_ld)
            pltpu.async_copy(*echunk(n2n_val, ch + 1, val_v.at[1 - b]), sem_ld)
            for k in range(_CHW):
                pltpu.make_async_copy(val_v.at[b, k], acc_s.at[si_v.at[b, k]],
                                      sem_st).wait()
        return carry

    lax.fori_loop(0, _NCH // 2, p1, 0)
    pltpu.make_async_copy(*echunk(n2n_row, _NCH, si_v.at[0]), sem_ld).wait()
    pltpu.make_async_copy(*echunk(n2n_val, _NCH, val_v.at[0]), sem_ld).wait()

    for j in range(_SGR // (16 * _CHW)):
        off = pl.multiple_of(sbase + j * _CHW, _CHW)
        pltpu.sync_copy(subg_row.at[ci, pl.ds(off, _CHW)], si_v.at[0])
        pltpu.sync_copy(subg_val.at[ci, pl.ds(off, _CHW)], val_v.at[0])
        for k in range(_CHW):
            pltpu.sync_copy(val_v.at[0, k], acc_s.at[si_v.at[0, k]], add=True)
    plsc.subcore_barrier()

    pltpu.async_copy(*echunk(sage_src, 0, si_v.at[0]), sem_ld)
    pltpu.async_copy(*echunk(sage_dst, 0, di_v.at[0]), sem_ld)

    def p2(j, carry):
        for b in range(2):
            ch = 2 * j + b
            pltpu.make_async_copy(*echunk(sage_src, ch, si_v.at[b]), sem_ld).wait()
            pltpu.make_async_copy(*echunk(sage_dst, ch, di_v.at[b]), sem_ld).wait()
            for k in range(_CHW):
                pltpu.async_copy(acc_s.at[si_v.at[b, k]], g_v.at[k], sem_g)
            for k in range(_CHW):
                pltpu.make_async_copy(acc_s.at[si_v.at[b, k]], g_v.at[k],
                                      sem_g).wait()
            for k in range(_CHW):
                pltpu.async_copy(g_v.at[k], acc_ts.at[di_v.at[b, k]],
                                 sem_st, add=True)
                pltpu.async_copy(ones_v, acc_cn.at[di_v.at[b, k]],
                                 sem_st, add=True)
            pltpu.async_copy(*echunk(sage_src, ch + 1, si_v.at[1 - b]), sem_ld)
            pltpu.async_copy(*echunk(sage_dst, ch + 1, di_v.at[1 - b]), sem_ld)
            for k in range(_CHW):
                pltpu.make_async_copy(g_v.at[k], acc_ts.at[di_v.at[b, k]],
                                      sem_st).wait()
                pltpu.make_async_copy(ones_v, acc_cn.at[di_v.at[b, k]],
                                      sem_st).wait()
        return carry

    lax.fori_loop(0, _NCH // 2, p2, 0)
    pltpu.make_async_copy(*echunk(sage_src, _NCH, si_v.at[0]), sem_ld).wait()
    pltpu.make_async_copy(*echunk(sage_dst, _NCH, di_v.at[0]), sem_ld).wait()
    plsc.subcore_barrier()

    @pl.when(ci == 0)
    def _():
        pltpu.sync_copy(acc_s.at[slc], s0_out.at[slc])
        pltpu.sync_copy(acc_ts.at[slc], ts0_out.at[slc])
        pltpu.sync_copy(acc_cn.at[slc], cn0_out.at[slc])

    @pl.when(ci == 1)
    def _():
        pltpu.sync_copy(acc_s.at[slc], s1_out.at[slc])
        pltpu.sync_copy(acc_ts.at[slc], ts1_out.at[slc])
        pltpu.sync_copy(acc_cn.at[slc], cn1_out.at[slc])

    @pl.when(sid == 0)
    def _():
        pltpu.sync_copy(acols.at[pl.ds(ci * _LANES, _LANES)], ac_v)
        pltpu.sync_copy(acc_s.at[ac_v], sa_v)
        pltpu.sync_copy(acc_ts.at[ac_v], tsa_v)
        pltpu.sync_copy(acc_cn.at[ac_v], ca_v)

        @pl.when(ci == 0)
        def _():
            pltpu.sync_copy(sa_v, sa0_out)
            pltpu.sync_copy(tsa_v, tsa0_out)
            pltpu.sync_copy(ca_v, ca0_out)

        @pl.when(ci == 1)
        def _():
            pltpu.sync_copy(sa_v, sa1_out)
            pltpu.sync_copy(tsa_v, tsa1_out)
            pltpu.sync_copy(ca_v, ca1_out)


def _sc_segment_sums(n2n_row, n2n_val, subg_row, subg_val, sage_src, sage_dst,
                     zeros, acols):
    f32 = jnp.float32
    outs = ([jax.ShapeDtypeStruct((_NPAD,), f32)] * 6
            + [jax.ShapeDtypeStruct((_LANES,), f32)] * 6)
    scr = [
        pltpu.VMEM((2, _CHW, _LANES), jnp.int32),
        pltpu.VMEM((2, _CHW, _LANES), jnp.int32),
        pltpu.VMEM((2, _CHW, _LANES), f32),
        pltpu.VMEM((_CHW, _LANES), f32),
        pltpu.VMEM((_LANES,), f32),
        pltpu.VMEM((_LANES,), jnp.int32),
        pltpu.VMEM((_LANES,), f32),
        pltpu.VMEM((_LANES,), f32),
        pltpu.VMEM((_LANES,), f32),
        pltpu.VMEM_SHARED((_NPAD,), f32),
        pltpu.VMEM_SHARED((_NPAD,), f32),
        pltpu.VMEM_SHARED((_NPAD,), f32),
        pltpu.SemaphoreType.DMA,
        pltpu.SemaphoreType.DMA,
        pltpu.SemaphoreType.DMA,
    ]
    mesh = plsc.VectorSubcoreMesh(core_axis_name="c", subcore_axis_name="s")
    s0, s1, ts0, ts1, cn0, cn1, sa0, sa1, tsa0, tsa1, ca0, ca1 = pl.kernel(
        _sc_body, mesh=mesh, out_type=outs, scratch_types=scr)(
        n2n_row, n2n_val, subg_row, subg_val, sage_src, sage_dst, zeros, acols)
    return (jnp.stack([s0, s1]), jnp.stack([ts0, ts1]),
            jnp.stack([cn0, cn1]), jnp.stack([sa0, sa1]),
            jnp.stack([tsa0, tsa1]), jnp.stack([ca0, ca1]))


def _const_vecs(w_ref, p_ref):
    w = w_ref[...]
    c = jnp.maximum(w[0:1, :] + w[1:2, :], 0.0)
    c = c / jnp.maximum(jnp.sqrt(jnp.sum(c * c)), 1e-12)
    p = p_ref[...]
    pb = p.astype(jnp.bfloat16).astype(jnp.float32)
    u = jnp.dot(c, p, precision=_HI, preferred_element_type=jnp.float32)
    v = jnp.dot(c, pb, precision=_HI, preferred_element_type=jnp.float32)
    return c, u, v


def _node_rows_bf16(s, t, c, v, p_ref, ws_ref):
    x = (s * c).astype(jnp.bfloat16)
    xl = jnp.dot(x, p_ref[...].astype(jnp.bfloat16),
                 preferred_element_type=jnp.float32)
    neigh = t * v
    cc = jnp.concatenate([xl.astype(jnp.bfloat16),
                          neigh.astype(jnp.bfloat16)], axis=1)
    msg = jnp.maximum(
        jnp.dot(cc, ws_ref[...].astype(jnp.bfloat16),
                preferred_element_type=jnp.float32), 0.0)
    return msg


def _rownorm(msg):
    nrm = jnp.sqrt(jnp.sum(msg * msg, axis=1, keepdims=True))
    return msg / jnp.maximum(nrm, 1e-12)


def _dense_body(s_ref, ts_ref, c_ref, w_ref, p_ref, ws_ref, out_ref):
    c, _, v = _const_vecs(w_ref, p_ref)
    s = s_ref[0]
    t = ts_ref[0] / jnp.maximum(c_ref[0], 1.0)
    msg = _node_rows_bf16(s, t, c, v, p_ref, ws_ref)
    out_ref[0] = _rownorm(msg)


def _head_body(sy_ref, tsy_ref, cy_ref, sa_ref, tsa_ref, ca_ref, av_ref,
               aux_ref, w_ref, p_ref, ws_ref, h1_ref, h2_ref, cp_ref, q_ref):
    c, u, v = _const_vecs(w_ref, p_ref)
    wsb = ws_ref[...].astype(jnp.bfloat16)
    q = jnp.zeros((_Y, 1), jnp.float32)
    for l in range(2):
        sy = sy_ref[l]
        ty = tsy_ref[l] / jnp.maximum(cy_ref[l], 1.0)
        xl_y = sy * u
        cc_y = jnp.concatenate([xl_y.astype(jnp.bfloat16),
                                (ty * v).astype(jnp.bfloat16)], axis=1)
        ymsg = jnp.maximum(
            jnp.dot(cc_y, wsb, preferred_element_type=jnp.float32), 0.0)
        ypot = _rownorm(ymsg)
        sa = sa_ref[l]
        ta = tsa_ref[l] / jnp.maximum(ca_ref[l], 1.0)
        amsg = _node_rows_bf16(sa, ta, c, v, p_ref, ws_ref)
        ae = _rownorm(amsg) * av_ref[l]
        cj = jnp.dot(ypot, cp_ref[...], precision=_HI,
                     preferred_element_type=jnp.float32)
        embed = ae * cj
        hidden = jnp.maximum(
            jnp.dot(embed, h1_ref[...], precision=_HI,
                    preferred_element_type=jnp.float32), 0.0)
        q = q + jnp.dot(hidden, h2_ref[0:_REG, :], precision=_HI,
                        preferred_element_type=jnp.float32)
        q = q + jnp.dot(aux_ref[l], h2_ref[_REG:, :], precision=_HI,
                        preferred_element_type=jnp.float32)
    q_ref[...] = q


def kernel(node_input, n2n_index0, n2n_value0, n2n_index1, n2n_value1,
           subg_index0, subg_value0, subg_index1, subg_value1,
           action_index0, action_value0, action_index1, action_value1,
           aux_input, sage_edge0, sage_edge1, w_n2l, p_node_conv, W_sage,
           h1_weight, h2_weight, cross_product):
    f32 = jnp.float32

    def prep(x, rows, padval):
        n = rows * _LANES - x.shape[0]
        return jnp.reshape(jnp.pad(x, (0, n), constant_values=padval),
                           (rows, _LANES))

    n2n_row = jnp.stack([prep(n2n_index0[0], _EPR_A, _PADBIN),
                         prep(n2n_index1[0], _EPR_A, _PADBIN)])
    n2n_val = jnp.stack([prep(n2n_value0, _EPR_A, 0),
                         prep(n2n_value1, _EPR_A, 0)])
    subg_row = jnp.stack([prep(subg_index0[0] + _N, _SGR, _PADBIN),
                          prep(subg_index1[0] + _N, _SGR, _PADBIN)])
    subg_val = jnp.stack([prep(subg_value0, _SGR, 0),
                          prep(subg_value1, _SGR, 0)])
    sage_src_s = jnp.stack([prep(sage_edge0[0], _EPR_A, 0),
                            prep(sage_edge1[0], _EPR_A, 0)])
    sage_dst_s = jnp.stack([prep(sage_edge0[1], _EPR_A, _PADBIN),
                            prep(sage_edge1[1], _EPR_A, _PADBIN)])
    zeros = jnp.zeros((_NPAD,), f32)
    acols = jnp.concatenate([
        jnp.pad(action_index0[1], (0, _LANES - _Y)),
        jnp.pad(action_index1[1], (0, _LANES - _Y))])

    S, TS, CN, SaG, TSaG, CNaG = _sc_segment_sums(
        n2n_row, n2n_val, subg_row, subg_val, sage_src_s, sage_dst_s, zeros,
        acols)

    Sp = S[..., None]
    TSp = TS[..., None]
    CNp = CN[..., None]

    bspec_s = pl.BlockSpec((1, _BLK, 1), lambda l, j: (l, j, 0))

    def bspec_w(shape):
        return pl.BlockSpec(shape, lambda l, j: tuple(0 for _ in shape))

    cur_msg = pl.pallas_call(
        _dense_body,
        grid=(2, _NB),
        in_specs=[bspec_s, bspec_s, bspec_s,
                  bspec_w((2, _EMB)), bspec_w((_EMB, _EMB)),
                  bspec_w((2 * _EMB, _EMB))],
        out_specs=pl.BlockSpec((1, _BLK, _EMB), lambda l, j: (l, j, 0)),
        out_shape=jax.ShapeDtypeStruct((2, _N, _EMB), f32),
    )(Sp, TSp, CNp, w_n2l, p_node_conv, W_sage)

    Sy = S[:, _N:_NPY][..., None]
    TSy = TS[:, _N:_NPY][..., None]
    CNy = CN[:, _N:_NPY][..., None]
    Sa = SaG[:, :_Y][..., None]
    TSa = TSaG[:, :_Y][..., None]
    CNa = CNaG[:, :_Y][..., None]
    av = jnp.stack([action_value0, action_value1])[..., None]
    aux_t = jnp.transpose(aux_input, (1, 0, 2))

    q = pl.pallas_call(
        _head_body,
        out_shape=jax.ShapeDtypeStruct((_Y, 1), f32),
    )(Sy, TSy, CNy, Sa, TSa, CNa, av, aux_t, w_n2l, p_node_conv, W_sage,
      h1_weight, h2_weight, cross_product)
    return (q, cur_msg)
